# baseline (speedup 1.0000x reference)
import jax
import jax.numpy as jnp
from jax import lax
from jax.experimental import pallas as pl
from jax.experimental.pallas import tpu as pltpu
from jax.experimental.pallas import tpu_sc as plsc

PAD = 1
BSZ = 4
SEQ = 8192
D = 1024
NW = 32
WPR = NW // BSZ
CHUNK = SEQ // WPR
G = 32
T = CHUNK // G
L = 16
NBUF = 3


def _sc_body(ids_hbm, table_hbm, out_hbm, ids_v, idx_v, rows_v, fp_s, hp_s,
             gsem, wsem):
    c = lax.axis_index("c")
    s = lax.axis_index("s")
    w = c * (NW // 2) + s
    r = w // WPR
    k = w % WPR
    kstart = k * CHUNK

    pltpu.sync_copy(ids_hbm.at[pl.ds(r * SEQ, SEQ)], ids_v)

    lanes = lax.iota(jnp.int32, L)
    zeros = jnp.zeros((L,), jnp.int32)
    ones = jnp.ones((L,), jnp.int32)

    def count_body(j, acc):
        v = ids_v[pl.ds(j * L, L)]
        take = (v != PAD) & (j * L + lanes < kstart)
        return acc + jnp.where(take, ones, zeros)

    acc = lax.fori_loop(jnp.int32(0), jnp.int32(SEQ // L), count_body, zeros)
    base = jnp.sum(acc, dtype=jnp.int32)

    def pos_body(j, carry):
        v0 = ids_v[pl.ds(kstart + j * G, L)]
        m0 = v0 != PAD
        mi0 = jnp.where(m0, ones, zeros)
        c0 = plsc.cumsum(mi0) + carry
        idx_v[pl.ds(j * G, L)] = jnp.where(m0, c0, 0) + PAD
        s0 = jnp.sum(mi0, dtype=jnp.int32)

        v1 = ids_v[pl.ds(kstart + j * G + L, L)]
        m1 = v1 != PAD
        mi1 = jnp.where(m1, ones, zeros)
        c1 = plsc.cumsum(mi1) + (carry + s0)
        idx_v[pl.ds(j * G + L, L)] = jnp.where(m1, c1, 0) + PAD
        s1 = jnp.sum(mi1, dtype=jnp.int32)

        fp_s[j] = carry + 2
        hp_s[j] = jnp.where(s0 + s1 == G, 0, 1).astype(jnp.int32)
        return carry + s0 + s1

    lax.fori_loop(jnp.int32(0), jnp.int32(T), pos_body, base)

    out_base = w * CHUNK

    def start_fetch(t, b):
        hp = hp_s[t]

        @pl.when(hp == 0)
        def _():
            pltpu.async_copy(
                table_hbm.at[pl.ds(fp_s[t], G)],
                rows_v.at[jnp.int32(b)],
                gsem.at[jnp.int32(b)],
            )

        @pl.when(hp != 0)
        def _():
            pltpu.async_copy(
                table_hbm.at[idx_v.at[pl.ds(jnp.int32(t * G), G)]],
                rows_v.at[jnp.int32(b)],
                gsem.at[jnp.int32(b)],
            )

    def wait_fetch(b):
        pltpu.make_async_copy(
            table_hbm.at[pl.ds(jnp.int32(0), G)],
            rows_v.at[jnp.int32(b)],
            gsem.at[jnp.int32(b)],
        ).wait()

    def wout(t, b):
        return pltpu.async_copy(
            rows_v.at[jnp.int32(b)],
            out_hbm.at[pl.ds(out_base + t * G, G)],
            wsem.at[jnp.int32(b)],
        )

    wcp = [None] * NBUF
    start_fetch(0, 0)
    start_fetch(1, 1)
    for t in range(T):
        b = t % NBUF
        if t + 2 < T:
            nb = (t + 2) % NBUF
            if wcp[nb] is not None:
                wcp[nb].wait()
                wcp[nb] = None
            start_fetch(t + 2, nb)
        wait_fetch(b)
        wcp[b] = wout(t, b)
    for cp in wcp:
        if cp is not None:
            cp.wait()


@jax.jit
def _embed(ids_flat, table):
    mesh = plsc.VectorSubcoreMesh(core_axis_name="c", subcore_axis_name="s")
    fn = pl.kernel(
        _sc_body,
        out_type=jax.ShapeDtypeStruct((BSZ * SEQ, D), jnp.float32),
        mesh=mesh,
        scratch_types=[
            pltpu.VMEM((SEQ,), jnp.int32),
            pltpu.VMEM((CHUNK,), jnp.int32),
            pltpu.VMEM((NBUF, G, D), jnp.float32),
            pltpu.SMEM((T,), jnp.int32),
            pltpu.SMEM((T,), jnp.int32),
            pltpu.SemaphoreType.DMA((NBUF,)),
            pltpu.SemaphoreType.DMA((NBUF,)),
        ],
        compiler_params=pltpu.CompilerParams(
            needs_layout_passes=False,
            use_tc_tiling_on_sc=False,
        ),
    )
    return fn(ids_flat, table)


def kernel(input, weights):
    ids = input.reshape(-1).astype(jnp.int32)
    out = _embed(ids, weights.astype(jnp.float32))
    return out.reshape(BSZ, SEQ, D)


# speedup vs baseline: 2.3813x; 2.3813x over previous
import functools

import jax
import jax.numpy as jnp
from jax import lax
from jax.experimental import pallas as pl
from jax.experimental.pallas import tpu as pltpu
from jax.experimental.pallas import tpu_sc as plsc

PAD = 1
BSZ = 4
SEQ = 8192
D = 1024
NW = 32
WPR = NW // BSZ
CHUNK = SEQ // WPR
G = 16
L = 16
NBUF = 6


def _sc_body(ids_hbm, table_hbm, out_hbm, ids_v, idx_v, rows_v, gsem, wsem):
    c = lax.axis_index("c")
    s = lax.axis_index("s")
    w = c * (NW // 2) + s
    r = w // WPR
    k = w % WPR
    kstart = k * CHUNK

    pltpu.sync_copy(ids_hbm.at[pl.ds(r * SEQ, SEQ)], ids_v)

    lanes = lax.iota(jnp.int32, L)
    zeros = jnp.zeros((L,), jnp.int32)
    ones = jnp.ones((L,), jnp.int32)

    def count_body(j, acc):
        v = ids_v[pl.ds(j * L, L)]
        take = (v != PAD) & (j * L + lanes < kstart)
        return acc + jnp.where(take, ones, zeros)

    acc = lax.fori_loop(jnp.int32(0), jnp.int32(SEQ // L), count_body, zeros)
    base = jnp.sum(acc, dtype=jnp.int32)

    def pos_body(j, carry):
        v = ids_v[pl.ds(kstart + j * L, L)]
        m = v != PAD
        mi = jnp.where(m, ones, zeros)
        csum = plsc.cumsum(mi) + carry
        idx_v[pl.ds(j * L, L)] = jnp.where(m, csum, 0) + PAD
        return carry + jnp.sum(mi, dtype=jnp.int32)

    lax.fori_loop(jnp.int32(0), jnp.int32(CHUNK // L), pos_body, base)

    out_base = w * CHUNK
    T = CHUNK // G

    def g(t, b):
        return pltpu.async_copy(
            table_hbm.at[idx_v.at[pl.ds(jnp.int32(t * G), G)]],
            rows_v.at[jnp.int32(b)],
            gsem.at[jnp.int32(b)],
        )

    def wout(t, b):
        return pltpu.async_copy(
            rows_v.at[jnp.int32(b)],
            out_hbm.at[pl.ds(out_base + t * G, G)],
            wsem.at[jnp.int32(b)],
        )

    P = NBUF - 1
    cps = [None] * T
    wcp = [None] * NBUF
    for t in range(min(P, T)):
        cps[t] = g(t, t % NBUF)
    for t in range(T):
        b = t % NBUF
        if t + P < T:
            nb = (t + P) % NBUF
            if wcp[nb] is not None:
                wcp[nb].wait()
                wcp[nb] = None
            cps[t + P] = g(t + P, nb)
        cps[t].wait()
        wcp[b] = wout(t, b)
    for cp in wcp:
        if cp is not None:
            cp.wait()


@jax.jit
def _embed(ids_flat, table):
    mesh = plsc.VectorSubcoreMesh(core_axis_name="c", subcore_axis_name="s")
    fn = pl.kernel(
        _sc_body,
        out_type=jax.ShapeDtypeStruct((BSZ * SEQ, D), jnp.float32),
        mesh=mesh,
        scratch_types=[
            pltpu.VMEM((SEQ,), jnp.int32),
            pltpu.VMEM((CHUNK,), jnp.int32),
            pltpu.VMEM((NBUF, G, D), jnp.float32),
            pltpu.SemaphoreType.DMA((NBUF,)),
            pltpu.SemaphoreType.DMA((NBUF,)),
        ],
        compiler_params=pltpu.CompilerParams(needs_layout_passes=False),
    )
    return fn(ids_flat, table)


def kernel(input, weights):
    ids = input.reshape(-1).astype(jnp.int32)
    out = _embed(ids, weights.astype(jnp.float32))
    return out.reshape(BSZ, SEQ, D)
